# Initial kernel scaffold; baseline (speedup 1.0000x reference)
import jax
import jax.numpy as jnp
from jax.experimental import pallas as pl


def kernel(x, edge_index, Wl1, Wr1, att1, b1, g1, be1, Wl2, Wr2, att2, b2, g2, be2, Wl3, Wr3, att3, b3, g3, be3):
    raise NotImplementedError("write your pallas kernel here")


# speedup vs baseline: 7.0601x; 7.0601x over previous
import functools

import jax
import jax.numpy as jnp
from jax import lax
from jax.experimental import pallas as pl
from jax.experimental.pallas import tpu as pltpu
from jax.experimental.pallas import tpu_sc as plsc

N = 10000
F = 128
HID = 128
H = 8
C = 16
L = 16
NC = 2
NS = 16
NW = NC * NS
K = 128
NB = 320
NBA = 328
NPAD = NW * NB
CH = 90
CAPT = CH * K
RB = 1280
GN = NPAD // RB


def _lane_gather(t, idx):
    return lax.gather(
        t, idx[:, None],
        lax.GatherDimensionNumbers(offset_dims=(), collapsed_slice_dims=(0,),
                                   start_index_map=(0,)),
        (1,), mode=lax.GatherScatterMode.PROMISE_IN_BOUNDS)


def _make_sc_kernel():
    mesh = plsc.VectorSubcoreMesh(
        core_axis_name="c", subcore_axis_name="s", num_cores=NC,
        num_subcores=NS)

    @functools.partial(
        pl.kernel,
        out_type=(
            jax.ShapeDtypeStruct((NPAD, F), jnp.float32),
            jax.ShapeDtypeStruct((NPAD, L), jnp.float32),
        ),
        mesh=mesh,
        scratch_types=[
            pltpu.VMEM((K,), jnp.int32),
            pltpu.VMEM((K,), jnp.int32),
            pltpu.VMEM((K,), jnp.int32),
            pltpu.VMEM((K, F), jnp.float32),
            pltpu.VMEM((K, F), jnp.float32),
            pltpu.VMEM((NBA, F), jnp.float32),
            pltpu.VMEM((NBA, L), jnp.float32),
            pltpu.VMEM((H * C,), jnp.float32),
            pltpu.SemaphoreType.DMA,
            pltpu.SemaphoreType.DMA,
        ],
    )
    def sc_fn(xl_hbm, xr_hbm, att_hbm, src_hbm, dstg_hbm, dstl_hbm,
              acc_out, den_out, src_v, dstg_v, dstl_v, rows_l, rows_r,
              acc_v, den_v, attv, sem1, sem2):
        cid = lax.axis_index("c")
        sid = lax.axis_index("s")
        wid = cid * NS + sid
        z16 = jnp.zeros((L,), jnp.float32)

        pltpu.sync_copy(att_hbm, attv)

        def _zb(i, _):
            for j in range(F // L):
                acc_v[i, pl.ds(j * L, L)] = z16
            den_v[i, :] = z16
            return 0

        lax.fori_loop(0, NBA, _zb, 0)

        att_vecs = [attv[pl.ds(h * C, C)] for h in range(H)]
        lanes = lax.iota(jnp.int32, L)

        def _chunk(j, _):
            pltpu.sync_copy(src_hbm.at[wid, j], src_v)
            pltpu.sync_copy(dstg_hbm.at[wid, j], dstg_v)
            pltpu.sync_copy(dstl_hbm.at[wid, j], dstl_v)
            pltpu.async_copy(xl_hbm.at[src_v], rows_l, sem1).wait()
            pltpu.async_copy(xr_hbm.at[dstg_v], rows_r, sem2).wait()

            def _grp(gg, _):
                e0 = gg * L
                dl16 = dstl_v[pl.ds(e0, L)]
                for i in range(L):
                    e = e0 + i
                    dl = dl16[i]
                    wv = z16
                    for h in range(H):
                        zl = rows_l[e, pl.ds(h * C, C)]
                        zr = rows_r[e, pl.ds(h * C, C)]
                        z = zl + zr
                        z = jnp.maximum(z, 0.2 * z)
                        t = z * att_vecs[h]
                        for sh in (1, 2, 4, 8):
                            t = t + _lane_gather(t, lanes ^ sh)
                        wh = jnp.exp(t)
                        wv = jnp.where(lanes == h, wh, wv)
                        acc_v[dl, pl.ds(h * C, C)] = (
                            acc_v[dl, pl.ds(h * C, C)] + zl * wh)
                    den_v[dl, :] = den_v[dl, :] + wv
                return 0

            lax.fori_loop(0, K // L, _grp, 0)
            return 0

        lax.fori_loop(0, CH, _chunk, 0)

        g0 = wid * NB
        pltpu.sync_copy(acc_v.at[pl.ds(0, NB)], acc_out.at[pl.ds(g0, NB)])
        pltpu.sync_copy(den_v.at[pl.ds(0, NB)], den_out.at[pl.ds(g0, NB)])

    return sc_fn


def _pre_call(xp, wl, wr):
    def body(x_ref, wl_ref, wr_ref, xl_ref, xr_ref):
        xb = x_ref[...]
        xl_ref[...] = jnp.dot(xb, wl_ref[...],
                              preferred_element_type=jnp.float32)
        xr_ref[...] = jnp.dot(xb, wr_ref[...],
                              preferred_element_type=jnp.float32)

    return pl.pallas_call(
        body,
        grid=(GN,),
        in_specs=[
            pl.BlockSpec((RB, F), lambda i: (i, 0)),
            pl.BlockSpec((F, HID), lambda i: (0, 0)),
            pl.BlockSpec((F, HID), lambda i: (0, 0)),
        ],
        out_specs=[
            pl.BlockSpec((RB, HID), lambda i: (i, 0)),
            pl.BlockSpec((RB, HID), lambda i: (i, 0)),
        ],
        out_shape=[jax.ShapeDtypeStruct((NPAD, HID), jnp.float32)] * 2,
    )(xp, wl, wr)


def _mid_call(acc, den, b, g, be, resid, wl, wr, last):
    has_res = resid is not None

    def body(*refs):
        it = iter(refs)
        acc_ref = next(it)
        den_ref = next(it)
        b_ref = next(it)
        g_ref = next(it)
        be_ref = next(it)
        res_ref = next(it) if has_res else None
        wl_ref = None if last else next(it)
        wr_ref = None if last else next(it)
        y_ref = next(it)
        xl_ref = None if last else next(it)
        xr_ref = None if last else next(it)

        den16 = den_ref[...]
        ji = lax.broadcasted_iota(jnp.int32, (L, HID), 0)
        ci = lax.broadcasted_iota(jnp.int32, (L, HID), 1)
        em = jnp.where(ci // C == ji, 1.0, 0.0).astype(jnp.float32)
        dfull = jnp.dot(den16, em, preferred_element_type=jnp.float32)
        o = acc_ref[...] / (dfull + 1e-16) + b_ref[...]
        m = jnp.mean(o, axis=-1, keepdims=True)
        v = jnp.mean((o - m) ** 2, axis=-1, keepdims=True)
        yn = (o - m) / jnp.sqrt(v + 1e-5) * g_ref[...] + be_ref[...]
        y = jnp.where(yn > 0, yn, jnp.exp(jnp.minimum(yn, 0.0)) - 1.0)
        if has_res:
            y = y + res_ref[...]
        y_ref[...] = y
        if not last:
            xl_ref[...] = jnp.dot(y, wl_ref[...],
                                  preferred_element_type=jnp.float32)
            xr_ref[...] = jnp.dot(y, wr_ref[...],
                                  preferred_element_type=jnp.float32)

    in_specs = [
        pl.BlockSpec((RB, F), lambda i: (i, 0)),
        pl.BlockSpec((RB, L), lambda i: (i, 0)),
        pl.BlockSpec((1, HID), lambda i: (0, 0)),
        pl.BlockSpec((1, HID), lambda i: (0, 0)),
        pl.BlockSpec((1, HID), lambda i: (0, 0)),
    ]
    args = [acc, den, b, g, be]
    if has_res:
        in_specs.append(pl.BlockSpec((RB, F), lambda i: (i, 0)))
        args.append(resid)
    out_specs = [pl.BlockSpec((RB, HID), lambda i: (i, 0))]
    n_out = N if last else NPAD
    out_shape = [jax.ShapeDtypeStruct((n_out, HID), jnp.float32)]
    if not last:
        in_specs += [pl.BlockSpec((F, HID), lambda i: (0, 0))] * 2
        args += [wl, wr]
        out_specs += [pl.BlockSpec((RB, HID), lambda i: (i, 0))] * 2
        out_shape += [jax.ShapeDtypeStruct((NPAD, HID), jnp.float32)] * 2

    return pl.pallas_call(
        body,
        grid=(GN,),
        in_specs=in_specs,
        out_specs=out_specs,
        out_shape=out_shape,
    )(*args)


def kernel(x, edge_index, Wl1, Wr1, att1, b1, g1, be1, Wl2, Wr2, att2, b2,
           g2, be2, Wl3, Wr3, att3, b3, g3, be3):
    idt = edge_index.dtype
    loop = jnp.arange(N, dtype=idt)
    src = jnp.concatenate([edge_index[0], loop])
    dst = jnp.concatenate([edge_index[1], loop])

    bucket = dst // NB
    onehot = (bucket[None, :] == jnp.arange(NW, dtype=idt)[:, None]).astype(idt)
    ranks_all = jnp.cumsum(onehot, axis=1) - 1
    rank = ranks_all[bucket, jnp.arange(dst.shape[0])]
    pos = bucket * CAPT + rank
    pos = jnp.where(rank < CAPT, pos, NW * CAPT)
    src_p = jnp.zeros((NW * CAPT,), idt).at[pos].set(src, mode="drop")
    dstg_p = jnp.zeros((NW * CAPT,), idt).at[pos].set(dst, mode="drop")
    dstl_p = jnp.full((NW * CAPT,), NB, idt).at[pos].set(
        dst - bucket * NB, mode="drop")
    src3 = src_p.reshape(NW, CH, K)
    dstg3 = dstg_p.reshape(NW, CH, K)
    dstl3 = dstl_p.reshape(NW, CH, K)

    xp = jnp.pad(x, ((0, NPAD - N), (0, 0)))
    b1r, g1r, be1r = b1.reshape(1, HID), g1.reshape(1, HID), be1.reshape(1, HID)
    b2r, g2r, be2r = b2.reshape(1, HID), g2.reshape(1, HID), be2.reshape(1, HID)
    b3r, g3r, be3r = b3.reshape(1, HID), g3.reshape(1, HID), be3.reshape(1, HID)
    att1f, att2f, att3f = (att1.reshape(-1), att2.reshape(-1),
                           att3.reshape(-1))

    sc = _make_sc_kernel()

    xl, xr = _pre_call(xp, Wl1, Wr1)
    acc, den = sc(xl, xr, att1f, src3, dstg3, dstl3)
    y1, xl2, xr2 = _mid_call(acc, den, b1r, g1r, be1r, None, Wl2, Wr2, False)
    acc2, den2 = sc(xl2, xr2, att2f, src3, dstg3, dstl3)
    y2, xl3, xr3 = _mid_call(acc2, den2, b2r, g2r, be2r, y1, Wl3, Wr3, False)
    acc3, den3 = sc(xl3, xr3, att3f, src3, dstg3, dstl3)
    (y3,) = _mid_call(acc3, den3, b3r, g3r, be3r, y2, None, None, True)
    return y3
